# baseline (speedup 1.0000x reference)
import jax
import jax.numpy as jnp
from jax.experimental import pallas as pl

_L_BLOCK = 512


def _add_body(x_ref, t_ref, o_ref):
    o_ref[...] = x_ref[...] + t_ref[...][None, :, :]


def kernel(x, pos_table):
    B, L, D = x.shape
    lb = min(_L_BLOCK, L)
    return pl.pallas_call(
        _add_body,
        grid=(L // lb,),
        in_specs=[
            pl.BlockSpec((B, lb, D), lambda i: (0, i, 0)),
            pl.BlockSpec((lb, D), lambda i: (i, 0)),
        ],
        out_specs=pl.BlockSpec((B, lb, D), lambda i: (0, i, 0)),
        out_shape=jax.ShapeDtypeStruct((B, L, D), x.dtype),
        input_output_aliases={0: 0},
    )(x, pos_table[:L])


# speedup vs baseline: 1.8878x; 1.8878x over previous
import jax
import jax.numpy as jnp
from jax.experimental import pallas as pl

_L_BLOCK = 512


def _add_body(x_ref, t_ref, o_ref):
    o_ref[...] = x_ref[...] + t_ref[...][None, :, :]


def kernel(x, pos_table):
    B, L, D = x.shape
    lb = min(_L_BLOCK, L)
    return pl.pallas_call(
        _add_body,
        grid=(L // lb,),
        in_specs=[
            pl.BlockSpec((B, lb, D), lambda i: (0, i, 0)),
            pl.BlockSpec((lb, D), lambda i: (i, 0)),
        ],
        out_specs=pl.BlockSpec((B, lb, D), lambda i: (0, i, 0)),
        out_shape=jax.ShapeDtypeStruct((B, L, D), x.dtype),
    )(x, pos_table[:L])
